# baseline (speedup 1.0000x reference)
import functools

import jax
import jax.numpy as jnp
from jax import lax
from jax.experimental import pallas as pl
from jax.experimental.pallas import tpu as pltpu
from jax.experimental.pallas import tpu_sc as plsc

D_MODEL = 64
GROUP = 128
SCALE = 8.0
LANES = 16


@functools.lru_cache(maxsize=None)
def _build(num_groups, vocab):
    info = plsc.get_sparse_core_info()
    nw = info.num_cores * info.num_subcores
    g_per_w = num_groups // nw

    mesh = plsc.VectorSubcoreMesh(core_axis_name="c", subcore_axis_name="s")

    @functools.partial(
        pl.kernel,
        mesh=mesh,
        out_type=jax.ShapeDtypeStruct((num_groups * GROUP, D_MODEL), jnp.float32),
        scratch_types=[
            pltpu.VMEM((g_per_w, GROUP), jnp.int32),
            pltpu.VMEM((GROUP, D_MODEL), jnp.float32),
            pltpu.SemaphoreType.DMA,
        ],
        compiler_params=pltpu.CompilerParams(use_tc_tiling_on_sc=False),
    )
    def k(table_hbm, idx_hbm, out_hbm, idx_v, rows_v, gsem):
        wid = lax.axis_index("s") * info.num_cores + lax.axis_index("c")
        gbase = wid * g_per_w
        pltpu.sync_copy(idx_hbm.at[pl.ds(gbase, g_per_w)], idx_v)

        def group_body(g, carry):
            pltpu.async_copy(table_hbm.at[idx_v.at[g]], rows_v, gsem).wait()

            def row_body(r, c2):
                for c in range(D_MODEL // LANES):
                    sl = pl.ds(c * LANES, LANES)
                    rows_v[r, sl] = rows_v[r, sl] * SCALE
                return c2

            lax.fori_loop(0, GROUP, row_body, 0, unroll=4)
            pltpu.sync_copy(rows_v, out_hbm.at[pl.ds((gbase + g) * GROUP, GROUP)])
            return carry

        lax.fori_loop(0, g_per_w, group_body, 0)

    return k


def kernel(x, table):
    b, h = x.shape
    idx = x.reshape(-1, GROUP).astype(jnp.int32)
    out = _build(idx.shape[0], table.shape[0])(table, idx)
    return out.reshape(b, h, D_MODEL)


# speedup vs baseline: 1.1564x; 1.1564x over previous
import functools

import jax
import jax.numpy as jnp
from jax import lax
from jax.experimental import pallas as pl
from jax.experimental.pallas import tpu as pltpu
from jax.experimental.pallas import tpu_sc as plsc

D_MODEL = 64
GROUP = 100
K = 4
NBUF = 4
LOOK = 3
SCALE = 8.0
LANES = 16


@functools.lru_cache(maxsize=None)
def _build(num_groups, vocab):
    info = plsc.get_sparse_core_info()
    nw = info.num_cores * info.num_subcores
    g_per_w = num_groups // nw
    n_chunks = g_per_w // K
    rows_per_chunk = K * GROUP

    mesh = plsc.VectorSubcoreMesh(core_axis_name="c", subcore_axis_name="s")

    @functools.partial(
        pl.kernel,
        mesh=mesh,
        out_type=jax.ShapeDtypeStruct((num_groups * GROUP, D_MODEL), jnp.float32),
        scratch_types=[
            pltpu.VMEM((g_per_w, GROUP), jnp.int32),
            pltpu.VMEM((rows_per_chunk, D_MODEL), jnp.float32),
            pltpu.VMEM((rows_per_chunk, D_MODEL), jnp.float32),
            pltpu.VMEM((rows_per_chunk, D_MODEL), jnp.float32),
            pltpu.VMEM((rows_per_chunk, D_MODEL), jnp.float32),
            pltpu.SemaphoreType.DMA,
            pltpu.SemaphoreType.DMA,
        ],
        compiler_params=pltpu.CompilerParams(use_tc_tiling_on_sc=False),
    )
    def k(table_hbm, idx_hbm, out_hbm, idx_v, b0, b1, b2, b3, gsem, ssem):
        bufs = [b0, b1, b2, b3]
        wid = lax.axis_index("s") * info.num_cores + lax.axis_index("c")
        gbase = wid * g_per_w
        row0 = gbase * GROUP
        pltpu.sync_copy(idx_hbm.at[pl.ds(gbase, g_per_w)], idx_v)

        ghandles = {}
        shandles = {}

        def start_gathers(c):
            p = c % NBUF
            hs = []
            for j in range(K):
                hs.append(pltpu.async_copy(
                    table_hbm.at[idx_v.at[c * K + j]],
                    bufs[p].at[pl.ds(j * GROUP, GROUP)],
                    gsem))
            ghandles[c] = hs

        def scale_chunk(p):
            buf = bufs[p]

            def row_body(r, carry):
                for q in range(D_MODEL // LANES):
                    sl = pl.ds(q * LANES, LANES)
                    buf[r, sl] = buf[r, sl] * SCALE
                return carry

            lax.fori_loop(0, rows_per_chunk, row_body, 0, unroll=4)

        for c in range(LOOK):
            start_gathers(c)
        for c in range(n_chunks):
            p = c % NBUF
            for h in ghandles.pop(c):
                h.wait()
            scale_chunk(p)
            shandles[c] = pltpu.async_copy(
                bufs[p],
                out_hbm.at[pl.ds(row0 + c * rows_per_chunk, rows_per_chunk)],
                ssem)
            nxt = c + LOOK
            if nxt < n_chunks:
                prev_user = nxt - NBUF
                if prev_user >= 0:
                    shandles.pop(prev_user).wait()
                start_gathers(nxt)
        for c in sorted(shandles):
            shandles.pop(c).wait()

    return k


def kernel(x, table):
    b, h = x.shape
    idx = x.reshape(-1, GROUP).astype(jnp.int32)
    out = _build(idx.shape[0], table.shape[0])(table, idx)
    return out.reshape(b, h, D_MODEL)
